# baseline (speedup 1.0000x reference)
import functools
import jax
import jax.numpy as jnp
from jax import lax
from jax.experimental import pallas as pl
from jax.experimental.pallas import tpu as pltpu
from jax.experimental.pallas import tpu_sc as plsc

_MAX_REL = 32
_S = 512
_D = 768
_NROWS = 2 * _MAX_REL + 1
_NC = 2
_NS = 16
_CBLKS = 8
_RBLKS = 4
_CW = _D // _CBLKS
_RH = _S // _RBLKS
_L = 16
_CV = _CW // _L
_INV = 1.0 / _S


def _row_weight(r, j):
    if r == 0:
        return max(0, _S - _MAX_REL - j)
    if r == _NROWS - 1:
        return max(0, j - (_MAX_REL - 1))
    v = r - _MAX_REL
    return 1 if (v <= j and v >= j - (_S - 1)) else 0


def _tree_sum(terms):
    while len(terms) > 1:
        nxt = [terms[i] + terms[i + 1] for i in range(0, len(terms) - 1, 2)]
        if len(terms) % 2:
            nxt.append(terms[-1])
        terms = nxt
    return terms[0]


def _walk(tbl_v, out_v, out_hbm, c0, sems, j0):
    sls = [pl.ds(k * _L, _L) for k in range(_CV)]
    inv = jnp.float32(_INV)

    acc = []
    for k in range(_CV):
        terms = [tbl_v[r, sls[k]]
                 for r in range(_NROWS) if _row_weight(r, j0) == 1]
        for r in (0, _NROWS - 1):
            w = _row_weight(r, j0)
            if w > 1:
                terms.append(tbl_v[r, sls[k]] * jnp.float32(w))
        a = _tree_sum(terms) * inv
        out_v[0, sls[k]] = a
        acc.append(a)

    d_mid = [(tbl_v[_NROWS - 1, sls[k]] - tbl_v[0, sls[k]]) * inv for k in range(_CV)]

    nq = len(sems)
    qh = _RH // nq
    cps = []
    anchor = acc
    s_anchor = 0
    for s in range(1, _RH):
        j = j0 + s
        hi = min(j, _MAX_REL) + _MAX_REL
        lo = max(j - (_S - _MAX_REL), 0)
        if hi == _NROWS - 1 and lo == 0:
            n = jnp.float32(s - s_anchor)
            row = [anchor[k] + n * d_mid[k] for k in range(_CV)]
        else:
            row = [acc[k] + (tbl_v[hi, sls[k]] - tbl_v[lo, sls[k]]) * inv
                   for k in range(_CV)]
            anchor = row
            s_anchor = s
        for k in range(_CV):
            out_v[s, sls[k]] = row[k]
        acc = row
        if s % qh == qh - 1:
            q = s // qh
            cps.append(pltpu.async_copy(
                out_v.at[pl.ds(q * qh, qh)],
                out_hbm.at[pl.ds(j0 + q * qh, qh), pl.ds(c0, _CW)],
                sems[q],
            ))
    for cp in cps:
        cp.wait()


def _rpe_sc_body(table_hbm, out_hbm, tbl_v, out_v, *sems):
    wid = lax.axis_index("s") * _NC + lax.axis_index("c")
    cb = wid % _CBLKS
    rb = wid // _CBLKS
    c0 = cb * _CW

    pltpu.sync_copy(table_hbm.at[:, pl.ds(c0, _CW)], tbl_v)

    for blk in range(_RBLKS):
        @pl.when(rb == blk)
        def _():
            _walk(tbl_v, out_v, out_hbm, c0, sems, blk * _RH)


def kernel(seq_len, table):
    mesh = plsc.VectorSubcoreMesh(
        core_axis_name="c", subcore_axis_name="s", num_cores=_NC, num_subcores=_NS
    )
    rpe = functools.partial(
        pl.kernel,
        out_type=jax.ShapeDtypeStruct((_S, _D), jnp.float32),
        mesh=mesh,
        scratch_types=[
            pltpu.VMEM((_NROWS, _CW), jnp.float32),
            pltpu.VMEM((_RH, _CW), jnp.float32),
            pltpu.SemaphoreType.DMA,
            pltpu.SemaphoreType.DMA,
            pltpu.SemaphoreType.DMA,
            pltpu.SemaphoreType.DMA,
        ],
        compiler_params=pltpu.CompilerParams(use_tc_tiling_on_sc=False),
    )(_rpe_sc_body)
    return rpe(table)[None, :, :]


# speedup vs baseline: 1.3785x; 1.3785x over previous
import functools
import jax
import jax.numpy as jnp
from jax import lax
from jax.experimental import pallas as pl
from jax.experimental.pallas import tpu as pltpu
from jax.experimental.pallas import tpu_sc as plsc

_MAX_REL = 32
_S = 512
_D = 768
_NROWS = 2 * _MAX_REL + 1
_NC = 2
_NS = 16
_CBLKS = 16
_RBLKS = 2
_CW = _D // _CBLKS
_RH = _S // _RBLKS
_NQ = 4
_QH = _RH // _NQ
_L = 16
_CV = _CW // _L
_INV = 1.0 / _S


def _row_weight(r, j):
    if r == 0:
        return max(0, _S - _MAX_REL - j)
    if r == _NROWS - 1:
        return max(0, j - (_MAX_REL - 1))
    v = r - _MAX_REL
    return 1 if (v <= j and v >= j - (_S - 1)) else 0


def _is_edge(j):
    hi = min(j, _MAX_REL) + _MAX_REL
    lo = max(j - (_S - _MAX_REL), 0)
    return not (hi == _NROWS - 1 and lo == 0)


def _tree_sum(terms):
    while len(terms) > 1:
        nxt = [terms[i] + terms[i + 1] for i in range(0, len(terms) - 1, 2)]
        if len(terms) % 2:
            nxt.append(terms[-1])
        terms = nxt
    return terms[0]


def _walk(tbl_v, out_v, out_hbm, c0, sems, j0):
    sls = [pl.ds(k * _L, _L) for k in range(_CV)]
    inv = jnp.float32(_INV)

    acc = []
    for k in range(_CV):
        terms = [tbl_v[r, sls[k]]
                 for r in range(_NROWS) if _row_weight(r, j0) == 1]
        for r in (0, _NROWS - 1):
            w = _row_weight(r, j0)
            if w > 1:
                terms.append(tbl_v[r, sls[k]] * jnp.float32(w))
        a = _tree_sum(terms) * inv
        out_v[0, sls[k]] = a
        acc.append(a)

    d_mid = [(tbl_v[_NROWS - 1, sls[k]] - tbl_v[0, sls[k]]) * inv
             for k in range(_CV)]

    anchor, s_anchor = acc, 0
    cps = []
    for q in range(_NQ):
        a0 = max(1, q * _QH)
        b0 = (q + 1) * _QH
        s = a0
        while s < b0:
            if _is_edge(j0 + s):
                while s < b0 and _is_edge(j0 + s):
                    j = j0 + s
                    hi = min(j, _MAX_REL) + _MAX_REL
                    lo = max(j - (_S - _MAX_REL), 0)
                    row = [acc[k] + (tbl_v[hi, sls[k]] - tbl_v[lo, sls[k]]) * inv
                           for k in range(_CV)]
                    for k in range(_CV):
                        out_v[s, sls[k]] = row[k]
                    acc = row
                    s += 1
                anchor, s_anchor = acc, s - 1
            else:
                e = s
                while e < b0 and not _is_edge(j0 + e):
                    e += 1
                anc, s_anc = anchor, s_anchor

                def seg_body(t, carry, anc=anc, s_anc=s_anc):
                    n = (t - s_anc).astype(jnp.float32)
                    for k in range(_CV):
                        out_v[t, sls[k]] = anc[k] + n * d_mid[k]
                    return carry

                lax.fori_loop(s, e, seg_body, 0, unroll=8)
                nlast = jnp.float32(e - 1 - s_anchor)
                acc = [anchor[k] + nlast * d_mid[k] for k in range(_CV)]
                s = e
        cps.append(pltpu.async_copy(
            out_v.at[pl.ds(q * _QH, _QH)],
            out_hbm.at[pl.ds(j0 + q * _QH, _QH), pl.ds(c0, _CW)],
            sems[q],
        ))
    for cp in cps:
        cp.wait()


def _rpe_sc_body(table_hbm, out_hbm, tbl_v, out_v, *sems):
    wid = lax.axis_index("s") * _NC + lax.axis_index("c")
    cb = wid % _CBLKS
    rb = wid // _CBLKS
    c0 = cb * _CW

    pltpu.sync_copy(table_hbm.at[:, pl.ds(c0, _CW)], tbl_v)

    for blk in range(_RBLKS):
        @pl.when(rb == blk)
        def _():
            _walk(tbl_v, out_v, out_hbm, c0, sems, blk * _RH)


def kernel(seq_len, table):
    mesh = plsc.VectorSubcoreMesh(
        core_axis_name="c", subcore_axis_name="s", num_cores=_NC, num_subcores=_NS
    )
    rpe = functools.partial(
        pl.kernel,
        out_type=jax.ShapeDtypeStruct((_S, _D), jnp.float32),
        mesh=mesh,
        scratch_types=[
            pltpu.VMEM((_NROWS, _CW), jnp.float32),
            pltpu.VMEM((_RH, _CW), jnp.float32),
            pltpu.SemaphoreType.DMA,
            pltpu.SemaphoreType.DMA,
            pltpu.SemaphoreType.DMA,
            pltpu.SemaphoreType.DMA,
        ],
        compiler_params=pltpu.CompilerParams(use_tc_tiling_on_sc=False),
    )(_rpe_sc_body)
    return rpe(table)[None, :, :]


# speedup vs baseline: 1.4313x; 1.0383x over previous
import functools
import jax
import jax.numpy as jnp
from jax import lax
from jax.experimental import pallas as pl
from jax.experimental.pallas import tpu as pltpu
from jax.experimental.pallas import tpu_sc as plsc

_MAX_REL = 32
_S = 512
_D = 768
_NROWS = 2 * _MAX_REL + 1
_NC = 2
_NS = 16
_CBLKS = 16
_RBLKS = 2
_CW = _D // _CBLKS
_RH = _S // _RBLKS
_NQ = 4
_QH = _RH // _NQ
_L = 16
_CV = _CW // _L
_INV = 1.0 / _S


def _row_weight(r, j):
    if r == 0:
        return max(0, _S - _MAX_REL - j)
    if r == _NROWS - 1:
        return max(0, j - (_MAX_REL - 1))
    v = r - _MAX_REL
    return 1 if (v <= j and v >= j - (_S - 1)) else 0


def _is_edge(j):
    hi = min(j, _MAX_REL) + _MAX_REL
    lo = max(j - (_S - _MAX_REL), 0)
    return not (hi == _NROWS - 1 and lo == 0)


def _walk(tbl_v, out_v, out_hbm, c0, sems, j0):
    sls = [pl.ds(k * _L, _L) for k in range(_CV)]
    inv = jnp.float32(_INV)

    ones_rows = [r for r in range(_NROWS) if _row_weight(r, j0) == 1]
    rlo, rhi = ones_rows[0], ones_rows[-1] + 1
    assert ones_rows == list(range(rlo, rhi))

    def init_body(r, accv):
        return tuple(accv[k] + tbl_v[r, sls[k]] for k in range(_CV))

    base = lax.fori_loop(
        rlo, rhi, init_body,
        tuple(jnp.zeros((_L,), jnp.float32) for _ in range(_CV)),
        unroll=8,
    )
    acc = []
    for k in range(_CV):
        a = base[k]
        for r in (0, _NROWS - 1):
            w = _row_weight(r, j0)
            if w > 1:
                a = a + tbl_v[r, sls[k]] * jnp.float32(w)
        a = a * inv
        out_v[0, sls[k]] = a
        acc.append(a)

    d_mid = [(tbl_v[_NROWS - 1, sls[k]] - tbl_v[0, sls[k]]) * inv
             for k in range(_CV)]

    anchor, s_anchor = acc, 0
    cps = []
    for q in range(_NQ):
        a0 = max(1, q * _QH)
        b0 = (q + 1) * _QH
        s = a0
        while s < b0:
            if _is_edge(j0 + s):
                e = s
                while e < b0 and _is_edge(j0 + e):
                    e += 1
                hi_a, lo_a = (min(j0 + s, _MAX_REL) + _MAX_REL,
                              max(j0 + s - (_S - _MAX_REL), 0))
                hi_b, lo_b = (min(j0 + e - 1, _MAX_REL) + _MAX_REL,
                              max(j0 + e - 1 - (_S - _MAX_REL), 0))
                hi_varies = hi_a != hi_b
                fixed = [tbl_v[lo_a if hi_varies else hi_a, sls[k]] * inv
                         for k in range(_CV)]
                sgn = inv if hi_varies else -inv
                off = (hi_a - s) if hi_varies else (lo_a - s)

                def edge_body(t, accv, off=off, sgn=sgn, fixed=fixed,
                              hv=hi_varies):
                    r = t + off
                    nxt = []
                    for k in range(_CV):
                        var = tbl_v[r, sls[k]] * sgn
                        a = (accv[k] + var - fixed[k]) if hv \
                            else (accv[k] + var + fixed[k])
                        out_v[t, sls[k]] = a
                        nxt.append(a)
                    return tuple(nxt)

                acc = list(lax.fori_loop(s, e, edge_body, tuple(acc),
                                         unroll=4))
                s = e
                anchor, s_anchor = acc, s - 1
            else:
                e = s
                while e < b0 and not _is_edge(j0 + e):
                    e += 1
                anc, s_anc = anchor, s_anchor

                def seg_body(t, carry, anc=anc, s_anc=s_anc):
                    n = (t - s_anc).astype(jnp.float32)
                    for k in range(_CV):
                        out_v[t, sls[k]] = anc[k] + n * d_mid[k]
                    return carry

                lax.fori_loop(s, e, seg_body, 0, unroll=8)
                nlast = jnp.float32(e - 1 - s_anchor)
                acc = [anchor[k] + nlast * d_mid[k] for k in range(_CV)]
                s = e
        cps.append(pltpu.async_copy(
            out_v.at[pl.ds(q * _QH, _QH)],
            out_hbm.at[pl.ds(j0 + q * _QH, _QH), pl.ds(c0, _CW)],
            sems[q],
        ))
    for cp in cps:
        cp.wait()


def _rpe_sc_body(table_hbm, out_hbm, tbl_v, out_v, *sems):
    wid = lax.axis_index("s") * _NC + lax.axis_index("c")
    cb = wid % _CBLKS
    rb = wid // _CBLKS
    c0 = cb * _CW

    pltpu.sync_copy(table_hbm.at[:, pl.ds(c0, _CW)], tbl_v)

    for blk in range(_RBLKS):
        @pl.when(rb == blk)
        def _():
            _walk(tbl_v, out_v, out_hbm, c0, sems, blk * _RH)


def kernel(seq_len, table):
    mesh = plsc.VectorSubcoreMesh(
        core_axis_name="c", subcore_axis_name="s", num_cores=_NC, num_subcores=_NS
    )
    rpe = functools.partial(
        pl.kernel,
        out_type=jax.ShapeDtypeStruct((_S, _D), jnp.float32),
        mesh=mesh,
        scratch_types=[
            pltpu.VMEM((_NROWS, _CW), jnp.float32),
            pltpu.VMEM((_RH, _CW), jnp.float32),
            pltpu.SemaphoreType.DMA,
            pltpu.SemaphoreType.DMA,
            pltpu.SemaphoreType.DMA,
            pltpu.SemaphoreType.DMA,
        ],
        compiler_params=pltpu.CompilerParams(use_tc_tiling_on_sc=False),
    )(_rpe_sc_body)
    return rpe(table)[None, :, :]
